# baseline (speedup 1.0000x reference)
import jax
import jax.numpy as jnp
from jax import lax
from jax.experimental import pallas as pl
from jax.experimental.pallas import tpu as pltpu

_N = 471
_TOPK = 50
_ROWS = 4096
_BLK = 1024
_NBLK = _ROWS // _BLK
_QR = 64
_NS_ITERS = 12
_ROUNDS = 10


def _body(x_ref, qt_ref, w_ref, linw_ref, linb_ref,
          alpha_ref, lamda_ref, rho_ref, mu_ref, out_ref, acc_ref):
    i = pl.program_id(0)

    @pl.when(i == 0)
    def _init():
        acc_ref[...] = jnp.zeros_like(acc_ref)

    xb = x_ref[...]
    rows = _BLK
    while rows > 8:
        rows //= 2
        xb = xb[:rows, :] + xb[rows:2 * rows, :]
    acc_ref[...] += xb

    @pl.when(i == _NBLK - 1)
    def _admm():
        alpha = alpha_ref[0]
        lamda = lamda_ref[0]
        rho = rho_ref[0]
        mu = mu_ref[0]
        w = w_ref[...]
        qt = qt_ref[...]
        a_mean = jnp.sum(acc_ref[...], axis=0, keepdims=True) * (1.0 / _ROWS)
        w2 = lax.dot_general(a_mean, linw_ref[...], (((1,), (1,)), ((), ())),
                             preferred_element_type=jnp.float32) + linb_ref[...]
        g = lax.dot_general(qt, qt, (((1,), (1,)), ((), ())),
                            preferred_element_type=jnp.float32)
        r = jnp.max(jnp.sum(jnp.abs(g), axis=1))
        xinv = g * (1.0 / (r * r))
        for _ in range(_NS_ITERS):
            gx = jnp.dot(g, xinv, preferred_element_type=jnp.float32)
            xinv = 2.0 * xinv - jnp.dot(xinv, gx,
                                        preferred_element_type=jnp.float32)
        p = alpha * jnp.dot(xinv, qt, preferred_element_type=jnp.float32)
        ones = jnp.ones_like(w)

        z = jnp.zeros_like(w)
        v = rho * w
        for _ in range(_ROUNDS):
            theta = lax.dot_general(v, p, (((1,), (1,)), ((), ())),
                                    preferred_element_type=jnp.float32)
            b = w + (1.0 / _N) * jnp.dot(theta, qt,
                                         preferred_element_type=jnp.float32)
            grad = (w2 + v + rho * (2.0 * z - b - w)
                    + (2.0 * lamda) * (jnp.sum(z) - 1.0) * ones)
            z2 = jnp.maximum(z - mu * grad, 0.0)
            zc = z2.reshape(_N, 1)
            rank = jnp.sum((zc > z2).astype(jnp.float32), axis=0, keepdims=True)
            z = jnp.where(rank < float(_TOPK), z2, 0.0)
            v = v + rho * (z - b)
        out_ref[...] = z / (jnp.sum(z) + 1e-8)


def kernel(x, q_t, w, b1, alpha, lamda, rho, mu, lin_W, lin_b):
    del b1
    w2d = w.reshape(1, _N)
    linb2d = lin_b.reshape(1, _N)
    smem = pl.BlockSpec(memory_space=pltpu.SMEM)
    out = pl.pallas_call(
        _body,
        grid=(_NBLK,),
        in_specs=[
            pl.BlockSpec((_BLK, _N), lambda i: (i, 0)),
            pl.BlockSpec((_QR, _N), lambda i: (0, 0)),
            pl.BlockSpec((1, _N), lambda i: (0, 0)),
            pl.BlockSpec((_N, _N), lambda i: (0, 0)),
            pl.BlockSpec((1, _N), lambda i: (0, 0)),
            smem, smem, smem, smem,
        ],
        out_specs=pl.BlockSpec((1, _N), lambda i: (0, 0)),
        out_shape=jax.ShapeDtypeStruct((1, _N), jnp.float32),
        scratch_shapes=[pltpu.VMEM((8, _N), jnp.float32)],
    )(x, q_t, w2d, lin_W, linb2d, alpha, lamda, rho, mu)
    return out.reshape(_N)
